# baseline (speedup 1.0000x reference)
import functools

import jax
import jax.numpy as jnp
from jax import lax
from jax.experimental import pallas as pl
from jax.experimental.pallas import tpu as pltpu
from jax.experimental.pallas import tpu_sc as plsc

N = 1024
E = 16384
H = 512
C = 64
XPAD = 512
AUGW = 640

F32 = jnp.float32
I32 = jnp.int32

NC = 2
NS = 16
NW = NC * NS
COLS = N // NW
EPW = E // NW


def _sc_mesh():
    return plsc.VectorSubcoreMesh(
        core_axis_name="c", subcore_axis_name="s",
        num_cores=NC, num_subcores=NS)


_SC_PARAMS = pltpu.CompilerParams(needs_layout_passes=False)


def _relu(x):
    return jnp.maximum(x, 0.0)


def _pack2(a, b):
    au = lax.bitcast_convert_type(a.astype(jnp.bfloat16), jnp.uint16)
    bu = lax.bitcast_convert_type(b.astype(jnp.bfloat16), jnp.uint16)
    w = (bu.astype(jnp.uint32) << 16) | au.astype(jnp.uint32)
    return lax.bitcast_convert_type(w, F32)


def _unpack2(w):
    u = lax.bitcast_convert_type(w, jnp.uint32)
    a = lax.bitcast_convert_type((u & 0xFFFF).astype(jnp.uint16),
                                 jnp.bfloat16).astype(F32)
    b = lax.bitcast_convert_type((u >> 16).astype(jnp.uint16),
                                 jnp.bfloat16).astype(F32)
    return a, b


_CBLK = 2048


def _tc_node1_body(dstrow, conf, xp,
                   mu1, mu1b, mu2, mu2b,
                   sg1, sg1row, sg1b, sg2, sg2b,
                   m1mu, m1sg, m1b,
                   mu_o, cmsg_o, cs_o):
    i = pl.program_id(0)
    ng = pl.num_programs(0)
    ohT = (lax.broadcasted_iota(I32, (N, _CBLK), 0)
           == dstrow[:]).astype(F32)
    vals = jnp.concatenate(
        [jnp.ones((_CBLK, 1), F32), conf[:], jnp.zeros((_CBLK, 126), F32)],
        axis=1)
    contrib = jnp.dot(ohT, vals, preferred_element_type=F32)

    @pl.when(i == 0)
    def _():
        cs_o[:] = contrib

    @pl.when(i > 0)
    def _():
        cs_o[:] = cs_o[:] + contrib

    @pl.when(i == ng - 1)
    def _():
        x = xp[:]
        h = _relu(jnp.dot(x, mu1[:], preferred_element_type=F32) + mu1b[:])
        mu = jnp.dot(h, mu2[:], preferred_element_type=F32) + mu2b[:]
        mu_o[:] = mu
        cntv = cs_o[:, 0:1]
        seed = jnp.where(cntv == 0.0, 1.0,
                         1.0 - cs_o[:, 1:2] / jnp.maximum(cntv, 1.0))
        hs = _relu(jnp.dot(x, sg1[:], preferred_element_type=F32)
                   + seed * sg1row[:] + sg1b[:])
        sigma = jax.nn.softplus(
            jnp.dot(hs, sg2[:], preferred_element_type=F32) + sg2b[:])
        cmsg_o[:] = (jnp.dot(mu, m1mu[:], preferred_element_type=F32)
                     + jnp.dot(sigma, m1sg[:], preferred_element_type=F32)
                     + m1b[:])


def _tc_node1(dst, conf, xp, p):
    outs = (
        jax.ShapeDtypeStruct((N, H), F32),
        jax.ShapeDtypeStruct((N, H), F32),
        jax.ShapeDtypeStruct((N, 128), F32),
    )
    g = E // _CBLK
    return pl.pallas_call(
        _tc_node1_body,
        grid=(g,),
        in_specs=[
            pl.BlockSpec((1, _CBLK), lambda i: (0, i)),
            pl.BlockSpec((_CBLK, 1), lambda i: (i, 0)),
            pl.BlockSpec((N, XPAD), lambda i: (0, 0)),
            pl.BlockSpec((XPAD, H), lambda i: (0, 0)),
            pl.BlockSpec((1, H), lambda i: (0, 0)),
            pl.BlockSpec((H, H), lambda i: (0, 0)),
            pl.BlockSpec((1, H), lambda i: (0, 0)),
            pl.BlockSpec((XPAD, H), lambda i: (0, 0)),
            pl.BlockSpec((1, H), lambda i: (0, 0)),
            pl.BlockSpec((1, H), lambda i: (0, 0)),
            pl.BlockSpec((H, H), lambda i: (0, 0)),
            pl.BlockSpec((1, H), lambda i: (0, 0)),
            pl.BlockSpec((H, H), lambda i: (0, 0)),
            pl.BlockSpec((H, H), lambda i: (0, 0)),
            pl.BlockSpec((1, H), lambda i: (0, 0)),
        ],
        out_specs=[
            pl.BlockSpec((N, H), lambda i: (0, 0)),
            pl.BlockSpec((N, H), lambda i: (0, 0)),
            pl.BlockSpec((N, 128), lambda i: (0, 0)),
        ],
        out_shape=outs,
    )(dst[None], conf, xp,
      p['mu1_wp'], p['mu1_b'][None], p['mu2_w'], p['mu2_b'][None],
      p['sg1_wp'], p['sg1_row'][None], p['sg1_b'][None], p['sg2_w'],
      p['sg2_b'][None],
      p['msg1_w'][:H], p['msg1_w'][H:2 * H], p['msg1_b'][None])


def _tc_resmm_body(avpf, p1_o, p2_o):
    rows = []
    for q in range(_AQ):
        rows.append(jnp.concatenate(
            [avpf[g * _AQ + q].reshape(_AR, _ACW) for g in range(_AG)],
            axis=1))
    a = jnp.concatenate(rows, axis=0)
    m = (a > 0.0).astype(F32)
    av = jnp.where(a > 0.0, a - 1.0, 0.0)
    p1_o[:] = (jnp.dot(av, m, preferred_element_type=F32)
               + jnp.dot(m, av, preferred_element_type=F32))
    p2_o[:] = jnp.dot(m, m, preferred_element_type=F32)


def _tc_resmm(avpf):
    outs = (
        jax.ShapeDtypeStruct((N, N), F32),
        jax.ShapeDtypeStruct((N, N), F32),
    )
    return pl.pallas_call(_tc_resmm_body, out_shape=outs)(
        avpf.reshape(NW, _AR * _ACW))


_EBLK = 2048


def _tc_edge_msg_body(cs, efp, wgt, res, dstrow, mef, m2, m2b, out):
    i = pl.program_id(0)
    h1 = _relu(cs[:] + jnp.dot(efp[:], mef[:], preferred_element_type=F32))
    msg = jnp.dot(h1, m2[:], preferred_element_type=F32) + m2b[:]
    w = wgt[:]
    vals = jnp.concatenate(
        [msg * w, w, res[:], jnp.zeros((_EBLK, AUGW - H - 2), F32)], axis=1)
    ohT = (lax.broadcasted_iota(I32, (N, _EBLK), 0)
           == dstrow[:]).astype(jnp.bfloat16)
    vhi = vals.astype(jnp.bfloat16)
    vlo = (vals - vhi.astype(F32)).astype(jnp.bfloat16)
    contrib = (jnp.dot(ohT, vhi, preferred_element_type=F32)
               + jnp.dot(ohT, vlo, preferred_element_type=F32))

    @pl.when(i == 0)
    def _():
        out[:] = contrib

    @pl.when(i > 0)
    def _():
        out[:] = out[:] + contrib


def _tc_edge_msg(cs, efp, wgt, res, dst, p):
    g = E // _EBLK
    return pl.pallas_call(
        _tc_edge_msg_body,
        grid=(g,),
        in_specs=[
            pl.BlockSpec((_EBLK, H), lambda i: (i, 0)),
            pl.BlockSpec((_EBLK, 8), lambda i: (i, 0)),
            pl.BlockSpec((_EBLK, 1), lambda i: (i, 0)),
            pl.BlockSpec((_EBLK, 1), lambda i: (i, 0)),
            pl.BlockSpec((1, _EBLK), lambda i: (0, i)),
            pl.BlockSpec((8, H), lambda i: (0, 0)),
            pl.BlockSpec((H, H), lambda i: (0, 0)),
            pl.BlockSpec((1, H), lambda i: (0, 0)),
        ],
        out_specs=pl.BlockSpec((N, AUGW), lambda i: (0, 0)),
        out_shape=jax.ShapeDtypeStruct((N, AUGW), F32),
    )(cs, efp, wgt, res, dst[None], p['mef_wp'], p['msg2_w'], p['msg2_b'][None])


def _tc_node2_body(aggp, cnt, mu,
                   muu1, muu1b, muu2, muu2b,
                   sgu1, sgu1row, sgu1b, sgu2, sgu2b,
                   sem1a, sem1b_, sem1c, met1a, met1b_, met1c,
                   mun_o, sgn_o, pr_o, qs_o):
    s = aggp[:]
    wsum = s[:, H:H + 1]
    sumr = s[:, H + 1:H + 2]
    agg = s[:, :H] / jnp.maximum(wsum, 1e-08)
    h = _relu(jnp.dot(agg, muu1[:], preferred_element_type=F32) + muu1b[:])
    mu_new = mu[:] + jnp.dot(h, muu2[:], preferred_element_type=F32) + muu2b[:]
    mun_o[:] = mu_new
    mean_r = sumr / jnp.maximum(cnt[:], 1.0)
    hg = _relu(jnp.dot(agg, sgu1[:], preferred_element_type=F32)
               + mean_r * sgu1row[:] + sgu1b[:])
    sgn_o[:] = jax.nn.softplus(jnp.dot(hg, sgu2[:], preferred_element_type=F32)
                               + sgu2b[:])
    psem = jnp.dot(mu_new, sem1a[:], preferred_element_type=F32) + sem1b_[:]
    pmet = jnp.dot(mu_new, met1a[:], preferred_element_type=F32) + met1b_[:]
    qsem = jnp.dot(mu_new, sem1c[:], preferred_element_type=F32)
    qmet = jnp.dot(mu_new, met1c[:], preferred_element_type=F32)
    pr_o[:] = _pack2(psem, pmet)
    qs_o[:] = _pack2(qsem, qmet)


def _tc_node2(aggp, cnt, mu, p):
    outs = (
        jax.ShapeDtypeStruct((N, H), F32),
        jax.ShapeDtypeStruct((N, H), F32),
        jax.ShapeDtypeStruct((N, H), F32),
        jax.ShapeDtypeStruct((N, H), F32),
    )
    return pl.pallas_call(_tc_node2_body, out_shape=outs)(
        aggp, cnt, mu,
        p['muu1_w'], p['muu1_b'][None], p['muu2_w'], p['muu2_b'][None],
        p['sgu1_w'][:H], p['sgu1_w'][H][None], p['sgu1_b'][None],
        p['sgu2_w'], p['sgu2_b'][None],
        p['sem1_w'][:H], p['sem1_b'][None], p['sem1_w'][H:],
        p['met1_w'][:H], p['met1_b'][None], p['met1_w'][H:2 * H])


def _tc_edge_out_body(g1, g2, dist, sem2, sem2b, metrow, met2, met2b,
                      sem_o, dp_o):
    d = dist[:]
    a1, b1 = _unpack2(g1[:])
    a2, b2 = _unpack2(g2[:])
    hs = _relu(a1 + a2)
    sem_o[:] = jnp.dot(hs, sem2[:], preferred_element_type=F32) + sem2b[:]
    hm = _relu(b1 + b2 + d * metrow[:])
    corr = jnp.dot(hm, met2[:], preferred_element_type=F32) + met2b[:]
    dp_o[:] = d + corr


def _tc_edge_out(g1, g2, dist, p):
    blk = 2048
    g = E // blk
    return pl.pallas_call(
        _tc_edge_out_body,
        grid=(g,),
        in_specs=[
            pl.BlockSpec((blk, H), lambda i: (i, 0)),
            pl.BlockSpec((blk, H), lambda i: (i, 0)),
            pl.BlockSpec((blk, 1), lambda i: (i, 0)),
            pl.BlockSpec((H, C), lambda i: (0, 0)),
            pl.BlockSpec((1, C), lambda i: (0, 0)),
            pl.BlockSpec((1, H), lambda i: (0, 0)),
            pl.BlockSpec((H, 1), lambda i: (0, 0)),
            pl.BlockSpec((1, 1), lambda i: (0, 0)),
        ],
        out_specs=[
            pl.BlockSpec((blk, C), lambda i: (i, 0)),
            pl.BlockSpec((blk, 1), lambda i: (i, 0)),
        ],
        out_shape=(
            jax.ShapeDtypeStruct((E, C), F32),
            jax.ShapeDtypeStruct((E, 1), F32),
        ),
    )(g1, g2, dist, p['sem2_w'], p['sem2_b'][None],
      p['met1_w'][2 * H][None], p['met2_w'], p['met2_b'][None])


_BCH = 2048
_BNCH = E // _BCH


_AQ = 4
_AG = NW // _AQ
_AR = N // _AQ
_ACW = N // _AG


def _sc_build_body(src_h, dst_h, d_h, z_h, avp_o, avp_t, src_v, dst_v, d_v):
    c = lax.axis_index("c")
    s = lax.axis_index("s")
    wid = s * NC + c
    q = lax.rem(wid, _AQ)
    g0 = lax.div(wid, _AQ)
    r_lo = q * _AR
    col_lo = g0 * _ACW
    pltpu.sync_copy(z_h, avp_t)
    for k in range(_BNCH):
        pltpu.sync_copy(src_h.at[pl.ds(k * _BCH, _BCH)], src_v)
        pltpu.sync_copy(dst_h.at[pl.ds(k * _BCH, _BCH)], dst_v)
        pltpu.sync_copy(d_h.at[pl.ds(k * _BCH, _BCH)], d_v)

        def body(g, _):
            s16 = src_v[pl.ds(g * 16, 16)]
            d16 = dst_v[pl.ds(g * 16, 16)]
            dv = d_v[pl.ds(g * 16, 16)]
            m2 = ((s16 >= r_lo) & (s16 < r_lo + _AR)
                  & (d16 >= col_lo) & (d16 < col_lo + _ACW))
            flat = jnp.where(m2, (s16 - r_lo) * _ACW + (d16 - col_lo), 0)
            plsc.store_scatter(avp_t, [flat], dv + 1.0, mask=m2)
            return _
        lax.fori_loop(0, _BCH // 16, body, 0)
    pltpu.sync_copy(avp_t, avp_o.at[pl.ds(wid * _AR * _ACW, _AR * _ACW)])


def _sp_build(src, dst, d):
    fn = pl.kernel(
        _sc_build_body,
        out_type=jax.ShapeDtypeStruct((NW * _AR * _ACW,), F32),
        mesh=_sc_mesh(),
        scratch_types=[
            pltpu.VMEM((_AR * _ACW,), F32),
            pltpu.VMEM((_BCH,), I32),
            pltpu.VMEM((_BCH,), I32),
            pltpu.VMEM((_BCH,), F32),
        ],
        compiler_params=_SC_PARAMS)
    return fn(src, dst, d, jnp.zeros((_AR * _ACW,), F32))


_GCH = 128
_GN = EPW // _GCH


def _sc_edge_gather_body(src_h, dst_h, d_h, p1_h, p2_h, cmsg_h,
                         res_o, wgt_o, cs_o,
                         srcv, dstv, dv, idxf, p1r, p2r, resv, wgtv,
                         rows_v, sem):
    c = lax.axis_index("c")
    s = lax.axis_index("s")
    wid = s * NC + c
    base = wid * EPW
    pltpu.sync_copy(src_h.at[pl.ds(base, EPW)], srcv)
    pltpu.sync_copy(dst_h.at[pl.ds(base, EPW)], dstv)
    pltpu.sync_copy(d_h.at[pl.ds(base, EPW)], dv)

    def fbody(g, _):
        f = srcv[pl.ds(g * 16, 16)] * N + dstv[pl.ds(g * 16, 16)]
        idxf[pl.ds(g * 16, 16)] = f
        return _
    lax.fori_loop(0, EPW // 16, fbody, 0)
    for j in range(_GN):
        pltpu.async_copy(p1_h.at[idxf.at[pl.ds(j * _GCH, _GCH)]], p1r,
                         sem).wait()
        pltpu.async_copy(p2_h.at[idxf.at[pl.ds(j * _GCH, _GCH)]], p2r,
                         sem).wait()

        def rbody(g, _):
            p1g = p1r[pl.ds(g * 16, 16)]
            p2g = p2r[pl.ds(g * 16, 16)]
            dg = dv[pl.ds(j * _GCH + g * 16, 16)]
            mean = jnp.where(p2g > 0.0, p1g / jnp.maximum(p2g, 1.0), dg)
            r = jnp.abs(dg - mean)
            resv[pl.ds(j * _GCH + g * 16, 16)] = r
            wgtv[pl.ds(j * _GCH + g * 16, 16)] = jnp.exp(-r)
            return _
        lax.fori_loop(0, _GCH // 16, rbody, 0)
    pltpu.sync_copy(resv, res_o.at[pl.ds(base, EPW)])
    pltpu.sync_copy(wgtv, wgt_o.at[pl.ds(base, EPW)])
    for j in range(_GN):
        pltpu.async_copy(cmsg_h.at[srcv.at[pl.ds(j * _GCH, _GCH)]], rows_v,
                         sem).wait()
        pltpu.sync_copy(rows_v, cs_o.at[pl.ds(base + j * _GCH, _GCH)])


def _sp_edge_gather(p1m, p2m, cmsg, src, dst, d):
    fn = pl.kernel(
        _sc_edge_gather_body,
        out_type=(jax.ShapeDtypeStruct((E,), F32),
                  jax.ShapeDtypeStruct((E,), F32),
                  jax.ShapeDtypeStruct((E, H), F32)),
        mesh=_sc_mesh(),
        scratch_types=[
            pltpu.VMEM((EPW,), I32),
            pltpu.VMEM((EPW,), I32),
            pltpu.VMEM((EPW,), F32),
            pltpu.VMEM((EPW,), I32),
            pltpu.VMEM((_GCH,), F32),
            pltpu.VMEM((_GCH,), F32),
            pltpu.VMEM((EPW,), F32),
            pltpu.VMEM((EPW,), F32),
            pltpu.VMEM((_GCH, H), F32),
            pltpu.SemaphoreType.DMA,
        ],
        compiler_params=_SC_PARAMS)
    res, wgt, cs = fn(src, dst, d, p1m.reshape(N * N), p2m.reshape(N * N),
                      cmsg)
    return res[:, None], wgt[:, None], cs


_FCH = 64
_FN = EPW // _FCH


def _sc_final_gather_body(src_h, dst_h, pr_h, qs_h, g1_o, g2_o,
                          srcv, dstv, bufa, bufb, sema, semb):
    c = lax.axis_index("c")
    s = lax.axis_index("s")
    wid = s * NC + c
    base = wid * EPW
    pltpu.sync_copy(src_h.at[pl.ds(base, EPW)], srcv)
    pltpu.sync_copy(dst_h.at[pl.ds(base, EPW)], dstv)
    for j in range(_FN):
        cpa = pltpu.async_copy(pr_h.at[srcv.at[pl.ds(j * _FCH, _FCH)]],
                               bufa, sema)
        cpb = pltpu.async_copy(qs_h.at[dstv.at[pl.ds(j * _FCH, _FCH)]],
                               bufb, semb)
        cpa.wait()
        pltpu.sync_copy(bufa, g1_o.at[pl.ds(base + j * _FCH, _FCH)])
        cpb.wait()
        pltpu.sync_copy(bufb, g2_o.at[pl.ds(base + j * _FCH, _FCH)])


def _sp_final_gather(pr, qs, src, dst):
    fn = pl.kernel(
        _sc_final_gather_body,
        out_type=(jax.ShapeDtypeStruct((E, H), F32),
                  jax.ShapeDtypeStruct((E, H), F32)),
        mesh=_sc_mesh(),
        scratch_types=[
            pltpu.VMEM((EPW,), I32),
            pltpu.VMEM((EPW,), I32),
            pltpu.VMEM((_FCH, H), F32),
            pltpu.VMEM((_FCH, H), F32),
            pltpu.SemaphoreType.DMA,
            pltpu.SemaphoreType.DMA,
        ],
        compiler_params=_SC_PARAMS)
    return fn(src, dst, pr, qs)


def _prep_params(params):
    p = dict(params)
    mu1 = jnp.zeros((XPAD, H), F32).at[:261].set(params['mu1_w'])
    sg1 = jnp.zeros((XPAD, H), F32).at[:261].set(params['sg1_w'][:261])
    p['mu1_wp'] = mu1
    p['sg1_wp'] = sg1
    p['sg1_row'] = params['sg1_w'][261]
    p['mef_wp'] = jnp.zeros((8, H), F32).at[:4].set(params['msg1_w'][2 * H:])
    return p


def kernel(node_sem, node_bbox, node_depth, edge_index, edge_dist, edge_conf,
           edge_angle, edge_depth_diff, params):
    src = edge_index[0]
    dst = edge_index[1]
    d = edge_dist[:, 0]
    p = _prep_params(params)

    xp = jnp.zeros((N, XPAD), F32).at[:, :261].set(
        jnp.concatenate([node_sem, node_bbox, node_depth], axis=-1))
    efp = jnp.zeros((E, 8), F32).at[:, :4].set(
        jnp.concatenate([edge_dist, edge_conf, edge_angle, edge_depth_diff],
                        axis=-1))

    avpf = _sp_build(src, dst, d)
    mu, cmsg, cntsc = _tc_node1(dst, edge_conf, xp, p)
    cnt = cntsc[:, 0:1]
    p1m, p2m = _tc_resmm(avpf)
    res, wgt, cs = _sp_edge_gather(p1m, p2m, cmsg, src, dst, d)
    agg_aug = _tc_edge_msg(cs, efp, wgt, res, dst, p)
    mu_new, sigma_new, pr, qs = _tc_node2(agg_aug, cnt, mu, p)
    g1, g2 = _sp_final_gather(pr, qs, src, dst)
    sem_logits, dist_pred = _tc_edge_out(g1, g2, edge_dist, p)
    return sem_logits, dist_pred, mu_new, sigma_new, res


# speedup vs baseline: 1.1726x; 1.1726x over previous
import functools

import jax
import jax.numpy as jnp
from jax import lax
from jax.experimental import pallas as pl
from jax.experimental.pallas import tpu as pltpu
from jax.experimental.pallas import tpu_sc as plsc

N = 1024
E = 16384
H = 512
C = 64
XPAD = 512
AUGW = 640

F32 = jnp.float32
I32 = jnp.int32

NC = 2
NS = 16
NW = NC * NS
COLS = N // NW
EPW = E // NW


def _sc_mesh():
    return plsc.VectorSubcoreMesh(
        core_axis_name="c", subcore_axis_name="s",
        num_cores=NC, num_subcores=NS)


_SC_PARAMS = pltpu.CompilerParams(needs_layout_passes=False)


def _relu(x):
    return jnp.maximum(x, 0.0)


def _pack2(a, b):
    au = lax.bitcast_convert_type(a.astype(jnp.bfloat16), jnp.uint16)
    bu = lax.bitcast_convert_type(b.astype(jnp.bfloat16), jnp.uint16)
    w = (bu.astype(jnp.uint32) << 16) | au.astype(jnp.uint32)
    return lax.bitcast_convert_type(w, F32)


def _unpack2(w):
    u = lax.bitcast_convert_type(w, jnp.uint32)
    a = lax.bitcast_convert_type((u & 0xFFFF).astype(jnp.uint16),
                                 jnp.bfloat16).astype(F32)
    b = lax.bitcast_convert_type((u >> 16).astype(jnp.uint16),
                                 jnp.bfloat16).astype(F32)
    return a, b


_CBLK = 2048


def _tc_node1_body(dstrow, conf, xp,
                   mu1, mu1b, mu2, mu2b,
                   sg1, sg1row, sg1b, sg2, sg2b,
                   m1mu, m1sg, m1b,
                   mu_o, cmsg_o, cs_o):
    i = pl.program_id(0)
    ng = pl.num_programs(0)
    ohT = (lax.broadcasted_iota(I32, (N, _CBLK), 0)
           == dstrow[:]).astype(F32)
    vals = jnp.concatenate(
        [jnp.ones((_CBLK, 1), F32), conf[:], jnp.zeros((_CBLK, 126), F32)],
        axis=1)
    contrib = jnp.dot(ohT, vals, preferred_element_type=F32)

    @pl.when(i == 0)
    def _():
        cs_o[:] = contrib

    @pl.when(i > 0)
    def _():
        cs_o[:] = cs_o[:] + contrib

    @pl.when(i == ng - 1)
    def _():
        x = xp[:]
        h = _relu(jnp.dot(x, mu1[:], preferred_element_type=F32) + mu1b[:])
        mu = jnp.dot(h, mu2[:], preferred_element_type=F32) + mu2b[:]
        mu_o[:] = mu
        cntv = cs_o[:, 0:1]
        seed = jnp.where(cntv == 0.0, 1.0,
                         1.0 - cs_o[:, 1:2] / jnp.maximum(cntv, 1.0))
        hs = _relu(jnp.dot(x, sg1[:], preferred_element_type=F32)
                   + seed * sg1row[:] + sg1b[:])
        sigma = jax.nn.softplus(
            jnp.dot(hs, sg2[:], preferred_element_type=F32) + sg2b[:])
        cmsg_o[:] = (jnp.dot(mu, m1mu[:], preferred_element_type=F32)
                     + jnp.dot(sigma, m1sg[:], preferred_element_type=F32)
                     + m1b[:])


def _tc_node1(dst, conf, xp, p):
    outs = (
        jax.ShapeDtypeStruct((N, H), F32),
        jax.ShapeDtypeStruct((N, H), F32),
        jax.ShapeDtypeStruct((N, 128), F32),
    )
    g = E // _CBLK
    return pl.pallas_call(
        _tc_node1_body,
        grid=(g,),
        in_specs=[
            pl.BlockSpec((1, _CBLK), lambda i: (0, i)),
            pl.BlockSpec((_CBLK, 1), lambda i: (i, 0)),
            pl.BlockSpec((N, XPAD), lambda i: (0, 0)),
            pl.BlockSpec((XPAD, H), lambda i: (0, 0)),
            pl.BlockSpec((1, H), lambda i: (0, 0)),
            pl.BlockSpec((H, H), lambda i: (0, 0)),
            pl.BlockSpec((1, H), lambda i: (0, 0)),
            pl.BlockSpec((XPAD, H), lambda i: (0, 0)),
            pl.BlockSpec((1, H), lambda i: (0, 0)),
            pl.BlockSpec((1, H), lambda i: (0, 0)),
            pl.BlockSpec((H, H), lambda i: (0, 0)),
            pl.BlockSpec((1, H), lambda i: (0, 0)),
            pl.BlockSpec((H, H), lambda i: (0, 0)),
            pl.BlockSpec((H, H), lambda i: (0, 0)),
            pl.BlockSpec((1, H), lambda i: (0, 0)),
        ],
        out_specs=[
            pl.BlockSpec((N, H), lambda i: (0, 0)),
            pl.BlockSpec((N, H), lambda i: (0, 0)),
            pl.BlockSpec((N, 128), lambda i: (0, 0)),
        ],
        out_shape=outs,
    )(dst[None], conf, xp,
      p['mu1_wp'], p['mu1_b'][None], p['mu2_w'], p['mu2_b'][None],
      p['sg1_wp'], p['sg1_row'][None], p['sg1_b'][None], p['sg2_w'],
      p['sg2_b'][None],
      p['msg1_w'][:H], p['msg1_w'][H:2 * H], p['msg1_b'][None])


def _tc_resmm_body(avpf, p1_o, p2_o):
    rows = []
    for q in range(_AQ):
        rows.append(jnp.concatenate(
            [avpf[g * _AQ + q].reshape(_AR, _ACW) for g in range(_AG)],
            axis=1))
    a = jnp.concatenate(rows, axis=0)
    m = (a > 0.0).astype(F32)
    av = jnp.where(a > 0.0, a - 1.0, 0.0)
    p1_o[:] = (jnp.dot(av, m, preferred_element_type=F32)
               + jnp.dot(m, av, preferred_element_type=F32))
    p2_o[:] = jnp.dot(m, m, preferred_element_type=F32)


def _tc_resmm(avpf):
    outs = (
        jax.ShapeDtypeStruct((N, N), F32),
        jax.ShapeDtypeStruct((N, N), F32),
    )
    return pl.pallas_call(_tc_resmm_body, out_shape=outs)(
        avpf.reshape(NW, _AR * _ACW))


_EBLK = 2048


def _tc_edge_msg_body(cs, efp, wgt, res, dstrow, mef, m2, m2b, out):
    i = pl.program_id(0)
    h1 = _relu(cs[:] + jnp.dot(efp[:], mef[:], preferred_element_type=F32))
    msg = jnp.dot(h1, m2[:], preferred_element_type=F32) + m2b[:]
    w = wgt[:]
    vals = jnp.concatenate(
        [msg * w, w, res[:], jnp.zeros((_EBLK, AUGW - H - 2), F32)], axis=1)
    ohT = (lax.broadcasted_iota(I32, (N, _EBLK), 0)
           == dstrow[:]).astype(F32)
    contrib = jnp.dot(ohT, vals, preferred_element_type=F32)

    @pl.when(i == 0)
    def _():
        out[:] = contrib

    @pl.when(i > 0)
    def _():
        out[:] = out[:] + contrib


def _tc_edge_msg(cs, efp, wgt, res, dst, p):
    g = E // _EBLK
    return pl.pallas_call(
        _tc_edge_msg_body,
        grid=(g,),
        in_specs=[
            pl.BlockSpec((_EBLK, H), lambda i: (i, 0)),
            pl.BlockSpec((_EBLK, 8), lambda i: (i, 0)),
            pl.BlockSpec((_EBLK, 1), lambda i: (i, 0)),
            pl.BlockSpec((_EBLK, 1), lambda i: (i, 0)),
            pl.BlockSpec((1, _EBLK), lambda i: (0, i)),
            pl.BlockSpec((8, H), lambda i: (0, 0)),
            pl.BlockSpec((H, H), lambda i: (0, 0)),
            pl.BlockSpec((1, H), lambda i: (0, 0)),
        ],
        out_specs=pl.BlockSpec((N, AUGW), lambda i: (0, 0)),
        out_shape=jax.ShapeDtypeStruct((N, AUGW), F32),
    )(cs, efp, wgt, res, dst[None], p['mef_wp'], p['msg2_w'], p['msg2_b'][None])


def _tc_node2_body(aggp, cnt, mu,
                   muu1, muu1b, muu2, muu2b,
                   sgu1, sgu1row, sgu1b, sgu2, sgu2b,
                   sem1a, sem1b_, sem1c, met1a, met1b_, met1c,
                   mun_o, sgn_o, pr_o, qs_o):
    s = aggp[:]
    wsum = s[:, H:H + 1]
    sumr = s[:, H + 1:H + 2]
    agg = s[:, :H] / jnp.maximum(wsum, 1e-08)
    h = _relu(jnp.dot(agg, muu1[:], preferred_element_type=F32) + muu1b[:])
    mu_new = mu[:] + jnp.dot(h, muu2[:], preferred_element_type=F32) + muu2b[:]
    mun_o[:] = mu_new
    mean_r = sumr / jnp.maximum(cnt[:], 1.0)
    hg = _relu(jnp.dot(agg, sgu1[:], preferred_element_type=F32)
               + mean_r * sgu1row[:] + sgu1b[:])
    sgn_o[:] = jax.nn.softplus(jnp.dot(hg, sgu2[:], preferred_element_type=F32)
                               + sgu2b[:])
    psem = jnp.dot(mu_new, sem1a[:], preferred_element_type=F32) + sem1b_[:]
    pmet = jnp.dot(mu_new, met1a[:], preferred_element_type=F32) + met1b_[:]
    qsem = jnp.dot(mu_new, sem1c[:], preferred_element_type=F32)
    qmet = jnp.dot(mu_new, met1c[:], preferred_element_type=F32)
    pr_o[:] = _pack2(psem, pmet)
    qs_o[:] = _pack2(qsem, qmet)


def _tc_node2(aggp, cnt, mu, p):
    outs = (
        jax.ShapeDtypeStruct((N, H), F32),
        jax.ShapeDtypeStruct((N, H), F32),
        jax.ShapeDtypeStruct((N, H), F32),
        jax.ShapeDtypeStruct((N, H), F32),
    )
    return pl.pallas_call(_tc_node2_body, out_shape=outs)(
        aggp, cnt, mu,
        p['muu1_w'], p['muu1_b'][None], p['muu2_w'], p['muu2_b'][None],
        p['sgu1_w'][:H], p['sgu1_w'][H][None], p['sgu1_b'][None],
        p['sgu2_w'], p['sgu2_b'][None],
        p['sem1_w'][:H], p['sem1_b'][None], p['sem1_w'][H:],
        p['met1_w'][:H], p['met1_b'][None], p['met1_w'][H:2 * H])


def _tc_edge_out_body(g1, g2, dist, sem2, sem2b, metrow, met2, met2b,
                      sem_o, dp_o):
    d = dist[:]
    a1, b1 = _unpack2(g1[:])
    a2, b2 = _unpack2(g2[:])
    hs = _relu(a1 + a2)
    sem_o[:] = jnp.dot(hs, sem2[:], preferred_element_type=F32) + sem2b[:]
    hm = _relu(b1 + b2 + d * metrow[:])
    corr = jnp.dot(hm, met2[:], preferred_element_type=F32) + met2b[:]
    dp_o[:] = d + corr


def _tc_edge_out(g1, g2, dist, p):
    blk = 2048
    g = E // blk
    return pl.pallas_call(
        _tc_edge_out_body,
        grid=(g,),
        in_specs=[
            pl.BlockSpec((blk, H), lambda i: (i, 0)),
            pl.BlockSpec((blk, H), lambda i: (i, 0)),
            pl.BlockSpec((blk, 1), lambda i: (i, 0)),
            pl.BlockSpec((H, C), lambda i: (0, 0)),
            pl.BlockSpec((1, C), lambda i: (0, 0)),
            pl.BlockSpec((1, H), lambda i: (0, 0)),
            pl.BlockSpec((H, 1), lambda i: (0, 0)),
            pl.BlockSpec((1, 1), lambda i: (0, 0)),
        ],
        out_specs=[
            pl.BlockSpec((blk, C), lambda i: (i, 0)),
            pl.BlockSpec((blk, 1), lambda i: (i, 0)),
        ],
        out_shape=(
            jax.ShapeDtypeStruct((E, C), F32),
            jax.ShapeDtypeStruct((E, 1), F32),
        ),
    )(g1, g2, dist, p['sem2_w'], p['sem2_b'][None],
      p['met1_w'][2 * H][None], p['met2_w'], p['met2_b'][None])


_BCH = 2048
_BNCH = E // _BCH


_AQ = 4
_AG = NW // _AQ
_AR = N // _AQ
_ACW = N // _AG


def _sc_build_body(src_h, dst_h, d_h, z_h, avp_o, avp_t,
                   src_v0, dst_v0, d_v0, src_v1, dst_v1, d_v1,
                   sem0, sem1):
    c = lax.axis_index("c")
    s = lax.axis_index("s")
    wid = s * NC + c
    q = lax.rem(wid, _AQ)
    g0 = lax.div(wid, _AQ)
    r_lo = q * _AR
    col_lo = g0 * _ACW
    zc = pltpu.async_copy(z_h, avp_t, sem1)
    bufs = [(src_v0, dst_v0, d_v0, sem0), (src_v1, dst_v1, d_v1, sem1)]

    def stage(k, b):
        sv, dv_, vv, sm = bufs[b]
        return (pltpu.async_copy(src_h.at[pl.ds(k * _BCH, _BCH)], sv, sm),
                pltpu.async_copy(dst_h.at[pl.ds(k * _BCH, _BCH)], dv_, sm),
                pltpu.async_copy(d_h.at[pl.ds(k * _BCH, _BCH)], vv, sm))

    pend = stage(0, 0)
    zc.wait()
    for k in range(_BNCH):
        b = k % 2
        for h in pend:
            h.wait()
        if k + 1 < _BNCH:
            nxt = stage(k + 1, (k + 1) % 2)
        sv, dvv, vv, _sm = bufs[b]

        def body(g, _):
            s16 = sv[pl.ds(g * 16, 16)]
            d16 = dvv[pl.ds(g * 16, 16)]
            dv = vv[pl.ds(g * 16, 16)]
            m2 = ((s16 >= r_lo) & (s16 < r_lo + _AR)
                  & (d16 >= col_lo) & (d16 < col_lo + _ACW))
            flat = jnp.where(m2, (s16 - r_lo) * _ACW + (d16 - col_lo), 0)
            plsc.store_scatter(avp_t, [flat], dv + 1.0, mask=m2)
            return _
        lax.fori_loop(0, _BCH // 16, body, 0)
        if k + 1 < _BNCH:
            pend = nxt
    pltpu.sync_copy(avp_t, avp_o.at[pl.ds(wid * _AR * _ACW, _AR * _ACW)])


def _sp_build(src, dst, d):
    fn = pl.kernel(
        _sc_build_body,
        out_type=jax.ShapeDtypeStruct((NW * _AR * _ACW,), F32),
        mesh=_sc_mesh(),
        scratch_types=[
            pltpu.VMEM((_AR * _ACW,), F32),
            pltpu.VMEM((_BCH,), I32),
            pltpu.VMEM((_BCH,), I32),
            pltpu.VMEM((_BCH,), F32),
            pltpu.VMEM((_BCH,), I32),
            pltpu.VMEM((_BCH,), I32),
            pltpu.VMEM((_BCH,), F32),
            pltpu.SemaphoreType.DMA,
            pltpu.SemaphoreType.DMA,
        ],
        compiler_params=_SC_PARAMS)
    return fn(src, dst, d, jnp.zeros((_AR * _ACW,), F32))


_GCH = 128
_GN = EPW // _GCH


_G2CH = 64
_G2N = EPW // _G2CH


def _sc_edge_gather_body(src_h, dst_h, d_h, p1_h, p2_h, cmsg_h,
                         res_o, wgt_o, cs_o,
                         srcv, dstv, dv, idxf, p1r, p2r, resv, wgtv,
                         rows0, rows1, sem, sem1, sem2, os0, os1):
    c = lax.axis_index("c")
    s = lax.axis_index("s")
    wid = s * NC + c
    base = wid * EPW
    h1 = pltpu.async_copy(src_h.at[pl.ds(base, EPW)], srcv, sem)
    h2 = pltpu.async_copy(dst_h.at[pl.ds(base, EPW)], dstv, sem)
    h3 = pltpu.async_copy(d_h.at[pl.ds(base, EPW)], dv, sem)
    h1.wait(); h2.wait(); h3.wait()

    rbufs = [rows0, rows1]
    gsems = [sem1, sem2]
    osems = [os0, os1]
    gh = [None, None]
    oh = [None, None]
    gh[0] = pltpu.async_copy(cmsg_h.at[srcv.at[pl.ds(0, _G2CH)]], rows0, sem1)

    def fbody(g, _):
        f = srcv[pl.ds(g * 16, 16)] * N + dstv[pl.ds(g * 16, 16)]
        idxf[pl.ds(g * 16, 16)] = f
        return _
    lax.fori_loop(0, EPW // 16, fbody, 0)
    hps = []
    for j in range(_GN):
        hps.append(pltpu.async_copy(
            p1_h.at[idxf.at[pl.ds(j * _GCH, _GCH)]],
            p1r.at[pl.ds(j * _GCH, _GCH)], sem))
        hps.append(pltpu.async_copy(
            p2_h.at[idxf.at[pl.ds(j * _GCH, _GCH)]],
            p2r.at[pl.ds(j * _GCH, _GCH)], sem))
    for h in hps:
        h.wait()

    def rbody(g, _):
        p1g = p1r[pl.ds(g * 16, 16)]
        p2g = p2r[pl.ds(g * 16, 16)]
        dg = dv[pl.ds(g * 16, 16)]
        mean = jnp.where(p2g > 0.0, p1g / jnp.maximum(p2g, 1.0), dg)
        r = jnp.abs(dg - mean)
        resv[pl.ds(g * 16, 16)] = r
        wgtv[pl.ds(g * 16, 16)] = jnp.exp(-r)
        return _
    lax.fori_loop(0, EPW // 16, rbody, 0)
    hr = pltpu.async_copy(resv, res_o.at[pl.ds(base, EPW)], sem)
    hw = pltpu.async_copy(wgtv, wgt_o.at[pl.ds(base, EPW)], sem)
    for j in range(_G2N):
        b = j % 2
        nb = (j + 1) % 2
        if j + 1 < _G2N:
            if oh[nb] is not None:
                oh[nb].wait()
            gh[nb] = pltpu.async_copy(
                cmsg_h.at[srcv.at[pl.ds((j + 1) * _G2CH, _G2CH)]],
                rbufs[nb], gsems[nb])
        gh[b].wait()
        oh[b] = pltpu.async_copy(
            rbufs[b], cs_o.at[pl.ds(base + j * _G2CH, _G2CH)], osems[b])
    oh[0].wait()
    oh[1].wait()
    hr.wait()
    hw.wait()


def _sp_edge_gather(p1m, p2m, cmsg, src, dst, d):
    fn = pl.kernel(
        _sc_edge_gather_body,
        out_type=(jax.ShapeDtypeStruct((E,), F32),
                  jax.ShapeDtypeStruct((E,), F32),
                  jax.ShapeDtypeStruct((E, H), F32)),
        mesh=_sc_mesh(),
        scratch_types=[
            pltpu.VMEM((EPW,), I32),
            pltpu.VMEM((EPW,), I32),
            pltpu.VMEM((EPW,), F32),
            pltpu.VMEM((EPW,), I32),
            pltpu.VMEM((EPW,), F32),
            pltpu.VMEM((EPW,), F32),
            pltpu.VMEM((EPW,), F32),
            pltpu.VMEM((EPW,), F32),
            pltpu.VMEM((_G2CH, H), F32),
            pltpu.VMEM((_G2CH, H), F32),
            pltpu.SemaphoreType.DMA,
            pltpu.SemaphoreType.DMA,
            pltpu.SemaphoreType.DMA,
            pltpu.SemaphoreType.DMA,
            pltpu.SemaphoreType.DMA,
        ],
        compiler_params=_SC_PARAMS)
    res, wgt, cs = fn(src, dst, d, p1m.reshape(N * N), p2m.reshape(N * N),
                      cmsg)
    return res[:, None], wgt[:, None], cs


_FCH = 32
_FN = EPW // _FCH


def _sc_final_gather_body(src_h, dst_h, pr_h, qs_h, g1_o, g2_o,
                          srcv, dstv, bufa0, bufa1, bufb0, bufb1,
                          ga0, ga1, gb0, gb1, oa0, oa1, ob0, ob1, ssem):
    c = lax.axis_index("c")
    s = lax.axis_index("s")
    wid = s * NC + c
    base = wid * EPW
    h1 = pltpu.async_copy(src_h.at[pl.ds(base, EPW)], srcv, ssem)
    h2 = pltpu.async_copy(dst_h.at[pl.ds(base, EPW)], dstv, ssem)
    h1.wait(); h2.wait()
    bufa = [bufa0, bufa1]
    bufb = [bufb0, bufb1]
    gsa = [ga0, ga1]
    gsb = [gb0, gb1]
    osa = [oa0, oa1]
    osb = [ob0, ob1]
    ga = [None, None]
    gb = [None, None]
    oa = [None, None]
    ob = [None, None]
    for j in range(_FN):
        b = j % 2
        if oa[b] is not None:
            oa[b].wait()
            ob[b].wait()
        ga[b] = pltpu.async_copy(pr_h.at[srcv.at[pl.ds(j * _FCH, _FCH)]],
                                 bufa[b], gsa[b])
        gb[b] = pltpu.async_copy(qs_h.at[dstv.at[pl.ds(j * _FCH, _FCH)]],
                                 bufb[b], gsb[b])
        if j >= 1:
            pb = (j - 1) % 2
            ga[pb].wait()
            oa[pb] = pltpu.async_copy(
                bufa[pb], g1_o.at[pl.ds(base + (j - 1) * _FCH, _FCH)],
                osa[pb])
            gb[pb].wait()
            ob[pb] = pltpu.async_copy(
                bufb[pb], g2_o.at[pl.ds(base + (j - 1) * _FCH, _FCH)],
                osb[pb])
    lb = (_FN - 1) % 2
    ga[lb].wait()
    pltpu.sync_copy(bufa[lb], g1_o.at[pl.ds(base + (_FN - 1) * _FCH, _FCH)])
    gb[lb].wait()
    pltpu.sync_copy(bufb[lb], g2_o.at[pl.ds(base + (_FN - 1) * _FCH, _FCH)])
    oa[(_FN - 2) % 2].wait()
    ob[(_FN - 2) % 2].wait()


def _sp_final_gather(pr, qs, src, dst):
    fn = pl.kernel(
        _sc_final_gather_body,
        out_type=(jax.ShapeDtypeStruct((E, H), F32),
                  jax.ShapeDtypeStruct((E, H), F32)),
        mesh=_sc_mesh(),
        scratch_types=[
            pltpu.VMEM((EPW,), I32),
            pltpu.VMEM((EPW,), I32),
            pltpu.VMEM((_FCH, H), F32),
            pltpu.VMEM((_FCH, H), F32),
            pltpu.VMEM((_FCH, H), F32),
            pltpu.VMEM((_FCH, H), F32),
            pltpu.SemaphoreType.DMA,
            pltpu.SemaphoreType.DMA,
            pltpu.SemaphoreType.DMA,
            pltpu.SemaphoreType.DMA,
            pltpu.SemaphoreType.DMA,
            pltpu.SemaphoreType.DMA,
            pltpu.SemaphoreType.DMA,
            pltpu.SemaphoreType.DMA,
            pltpu.SemaphoreType.DMA,
        ],
        compiler_params=_SC_PARAMS)
    return fn(src, dst, pr, qs)


def _prep_params(params):
    p = dict(params)
    mu1 = jnp.zeros((XPAD, H), F32).at[:261].set(params['mu1_w'])
    sg1 = jnp.zeros((XPAD, H), F32).at[:261].set(params['sg1_w'][:261])
    p['mu1_wp'] = mu1
    p['sg1_wp'] = sg1
    p['sg1_row'] = params['sg1_w'][261]
    p['mef_wp'] = jnp.zeros((8, H), F32).at[:4].set(params['msg1_w'][2 * H:])
    return p


def kernel(node_sem, node_bbox, node_depth, edge_index, edge_dist, edge_conf,
           edge_angle, edge_depth_diff, params):
    src = edge_index[0]
    dst = edge_index[1]
    d = edge_dist[:, 0]
    p = _prep_params(params)

    xp = jnp.zeros((N, XPAD), F32).at[:, :261].set(
        jnp.concatenate([node_sem, node_bbox, node_depth], axis=-1))
    efp = jnp.zeros((E, 8), F32).at[:, :4].set(
        jnp.concatenate([edge_dist, edge_conf, edge_angle, edge_depth_diff],
                        axis=-1))

    avpf = _sp_build(src, dst, d)
    mu, cmsg, cntsc = _tc_node1(dst, edge_conf, xp, p)
    cnt = cntsc[:, 0:1]
    p1m, p2m = _tc_resmm(avpf)
    res, wgt, cs = _sp_edge_gather(p1m, p2m, cmsg, src, dst, d)
    agg_aug = _tc_edge_msg(cs, efp, wgt, res, dst, p)
    mu_new, sigma_new, pr, qs = _tc_node2(agg_aug, cnt, mu, p)
    g1, g2 = _sp_final_gather(pr, qs, src, dst)
    sem_logits, dist_pred = _tc_edge_out(g1, g2, edge_dist, p)
    return sem_logits, dist_pred, mu_new, sigma_new, res


# speedup vs baseline: 1.1761x; 1.0030x over previous
import functools

import jax
import jax.numpy as jnp
from jax import lax
from jax.experimental import pallas as pl
from jax.experimental.pallas import tpu as pltpu
from jax.experimental.pallas import tpu_sc as plsc

N = 1024
E = 16384
H = 512
C = 64
XPAD = 512
AUGW = 640

F32 = jnp.float32
I32 = jnp.int32

NC = 2
NS = 16
NW = NC * NS
COLS = N // NW
EPW = E // NW


def _sc_mesh():
    return plsc.VectorSubcoreMesh(
        core_axis_name="c", subcore_axis_name="s",
        num_cores=NC, num_subcores=NS)


_SC_PARAMS = pltpu.CompilerParams(needs_layout_passes=False)


def _relu(x):
    return jnp.maximum(x, 0.0)


def _pack2(a, b):
    au = lax.bitcast_convert_type(a.astype(jnp.bfloat16), jnp.uint16)
    bu = lax.bitcast_convert_type(b.astype(jnp.bfloat16), jnp.uint16)
    w = (bu.astype(jnp.uint32) << 16) | au.astype(jnp.uint32)
    return lax.bitcast_convert_type(w, F32)


def _unpack2(w):
    u = lax.bitcast_convert_type(w, jnp.uint32)
    a = lax.bitcast_convert_type((u & 0xFFFF).astype(jnp.uint16),
                                 jnp.bfloat16).astype(F32)
    b = lax.bitcast_convert_type((u >> 16).astype(jnp.uint16),
                                 jnp.bfloat16).astype(F32)
    return a, b


_CBLK = 2048


def _tc_node1_body(dstrow, conf, xp,
                   mu1, mu1b, mu2, mu2b,
                   sg1, sg1row, sg1b, sg2, sg2b,
                   m1mu, m1sg, m1b,
                   mu_o, cmsg_o, cs_o):
    i = pl.program_id(0)
    ng = pl.num_programs(0)
    ohT = (lax.broadcasted_iota(I32, (N, _CBLK), 0)
           == dstrow[:]).astype(F32)
    vals = jnp.concatenate(
        [jnp.ones((_CBLK, 1), F32), conf[:], jnp.zeros((_CBLK, 126), F32)],
        axis=1)
    contrib = jnp.dot(ohT, vals, preferred_element_type=F32)

    @pl.when(i == 0)
    def _():
        cs_o[:] = contrib

    @pl.when(i > 0)
    def _():
        cs_o[:] = cs_o[:] + contrib

    @pl.when(i == ng - 1)
    def _():
        x = xp[:]
        h = _relu(jnp.dot(x, mu1[:], preferred_element_type=F32) + mu1b[:])
        mu = jnp.dot(h, mu2[:], preferred_element_type=F32) + mu2b[:]
        mu_o[:] = mu
        cntv = cs_o[:, 0:1]
        seed = jnp.where(cntv == 0.0, 1.0,
                         1.0 - cs_o[:, 1:2] / jnp.maximum(cntv, 1.0))
        hs = _relu(jnp.dot(x, sg1[:], preferred_element_type=F32)
                   + seed * sg1row[:] + sg1b[:])
        sigma = jax.nn.softplus(
            jnp.dot(hs, sg2[:], preferred_element_type=F32) + sg2b[:])
        cmsg_o[:] = (jnp.dot(mu, m1mu[:], preferred_element_type=F32)
                     + jnp.dot(sigma, m1sg[:], preferred_element_type=F32)
                     + m1b[:])


def _tc_node1(dst, conf, xp, p):
    outs = (
        jax.ShapeDtypeStruct((N, H), F32),
        jax.ShapeDtypeStruct((N, H), F32),
        jax.ShapeDtypeStruct((N, 128), F32),
    )
    g = E // _CBLK
    return pl.pallas_call(
        _tc_node1_body,
        grid=(g,),
        in_specs=[
            pl.BlockSpec((1, _CBLK), lambda i: (0, i)),
            pl.BlockSpec((_CBLK, 1), lambda i: (i, 0)),
            pl.BlockSpec((N, XPAD), lambda i: (0, 0)),
            pl.BlockSpec((XPAD, H), lambda i: (0, 0)),
            pl.BlockSpec((1, H), lambda i: (0, 0)),
            pl.BlockSpec((H, H), lambda i: (0, 0)),
            pl.BlockSpec((1, H), lambda i: (0, 0)),
            pl.BlockSpec((XPAD, H), lambda i: (0, 0)),
            pl.BlockSpec((1, H), lambda i: (0, 0)),
            pl.BlockSpec((1, H), lambda i: (0, 0)),
            pl.BlockSpec((H, H), lambda i: (0, 0)),
            pl.BlockSpec((1, H), lambda i: (0, 0)),
            pl.BlockSpec((H, H), lambda i: (0, 0)),
            pl.BlockSpec((H, H), lambda i: (0, 0)),
            pl.BlockSpec((1, H), lambda i: (0, 0)),
        ],
        out_specs=[
            pl.BlockSpec((N, H), lambda i: (0, 0)),
            pl.BlockSpec((N, H), lambda i: (0, 0)),
            pl.BlockSpec((N, 128), lambda i: (0, 0)),
        ],
        out_shape=outs,
    )(dst[None], conf, xp,
      p['mu1_wp'], p['mu1_b'][None], p['mu2_w'], p['mu2_b'][None],
      p['sg1_wp'], p['sg1_row'][None], p['sg1_b'][None], p['sg2_w'],
      p['sg2_b'][None],
      p['msg1_w'][:H], p['msg1_w'][H:2 * H], p['msg1_b'][None])


def _tc_resmm_body(avpf, p1_o, p2_o):
    rows = []
    for q in range(_AQ):
        rows.append(jnp.concatenate(
            [avpf[g * _AQ + q].reshape(_AR, _ACW) for g in range(_AG)],
            axis=1))
    a = jnp.concatenate(rows, axis=0)
    m = (a > 0.0).astype(F32)
    av = jnp.where(a > 0.0, a - 1.0, 0.0)
    p1_o[:] = (jnp.dot(av, m, preferred_element_type=F32)
               + jnp.dot(m, av, preferred_element_type=F32))
    m16 = m.astype(jnp.bfloat16)
    p2_o[:] = jnp.dot(m16, m16, preferred_element_type=F32)


def _tc_resmm(avpf):
    outs = (
        jax.ShapeDtypeStruct((N, N), F32),
        jax.ShapeDtypeStruct((N, N), F32),
    )
    return pl.pallas_call(_tc_resmm_body, out_shape=outs)(
        avpf.reshape(NW, _AR * _ACW))


_EBLK = 2048


def _tc_edge_msg_body(cs, efp, wgt, res, dstrow, mef, m2, m2b, out):
    i = pl.program_id(0)
    h1 = _relu(cs[:] + jnp.dot(efp[:], mef[:], preferred_element_type=F32))
    msg = jnp.dot(h1.astype(jnp.bfloat16), m2[:],
                  preferred_element_type=F32) + m2b[:]
    w = wgt[:]
    vals = jnp.concatenate(
        [msg * w, w, res[:], jnp.zeros((_EBLK, AUGW - H - 2), F32)], axis=1)
    ohT = (lax.broadcasted_iota(I32, (N, _EBLK), 0)
           == dstrow[:]).astype(jnp.bfloat16)
    contrib = jnp.dot(ohT, vals.astype(jnp.bfloat16),
                      preferred_element_type=F32)

    @pl.when(i == 0)
    def _():
        out[:] = contrib

    @pl.when(i > 0)
    def _():
        out[:] = out[:] + contrib


def _tc_edge_msg(cs, efp, wgt, res, dst, p):
    g = E // _EBLK
    return pl.pallas_call(
        _tc_edge_msg_body,
        grid=(g,),
        in_specs=[
            pl.BlockSpec((_EBLK, H), lambda i: (i, 0)),
            pl.BlockSpec((_EBLK, 8), lambda i: (i, 0)),
            pl.BlockSpec((_EBLK, 1), lambda i: (i, 0)),
            pl.BlockSpec((_EBLK, 1), lambda i: (i, 0)),
            pl.BlockSpec((1, _EBLK), lambda i: (0, i)),
            pl.BlockSpec((8, H), lambda i: (0, 0)),
            pl.BlockSpec((H, H), lambda i: (0, 0)),
            pl.BlockSpec((1, H), lambda i: (0, 0)),
        ],
        out_specs=pl.BlockSpec((N, AUGW), lambda i: (0, 0)),
        out_shape=jax.ShapeDtypeStruct((N, AUGW), F32),
    )(cs, efp, wgt, res, dst[None], p['mef_wp'],
      p['msg2_w'].astype(jnp.bfloat16), p['msg2_b'][None])


def _tc_node2_body(aggp, cnt, mu,
                   muu1, muu1b, muu2, muu2b,
                   sgu1, sgu1row, sgu1b, sgu2, sgu2b,
                   sem1a, sem1b_, sem1c, met1a, met1b_, met1c,
                   mun_o, sgn_o, pr_o, qs_o):
    s = aggp[:]
    wsum = s[:, H:H + 1]
    sumr = s[:, H + 1:H + 2]
    agg = s[:, :H] / jnp.maximum(wsum, 1e-08)
    h = _relu(jnp.dot(agg, muu1[:], preferred_element_type=F32) + muu1b[:])
    mu_new = mu[:] + jnp.dot(h, muu2[:], preferred_element_type=F32) + muu2b[:]
    mun_o[:] = mu_new
    mean_r = sumr / jnp.maximum(cnt[:], 1.0)
    hg = _relu(jnp.dot(agg, sgu1[:], preferred_element_type=F32)
               + mean_r * sgu1row[:] + sgu1b[:])
    sgn_o[:] = jax.nn.softplus(jnp.dot(hg, sgu2[:], preferred_element_type=F32)
                               + sgu2b[:])
    psem = jnp.dot(mu_new, sem1a[:], preferred_element_type=F32) + sem1b_[:]
    pmet = jnp.dot(mu_new, met1a[:], preferred_element_type=F32) + met1b_[:]
    qsem = jnp.dot(mu_new, sem1c[:], preferred_element_type=F32)
    qmet = jnp.dot(mu_new, met1c[:], preferred_element_type=F32)
    pr_o[:] = _pack2(psem, pmet)
    qs_o[:] = _pack2(qsem, qmet)


def _tc_node2(aggp, cnt, mu, p):
    outs = (
        jax.ShapeDtypeStruct((N, H), F32),
        jax.ShapeDtypeStruct((N, H), F32),
        jax.ShapeDtypeStruct((N, H), F32),
        jax.ShapeDtypeStruct((N, H), F32),
    )
    return pl.pallas_call(_tc_node2_body, out_shape=outs)(
        aggp, cnt, mu,
        p['muu1_w'], p['muu1_b'][None], p['muu2_w'], p['muu2_b'][None],
        p['sgu1_w'][:H], p['sgu1_w'][H][None], p['sgu1_b'][None],
        p['sgu2_w'], p['sgu2_b'][None],
        p['sem1_w'][:H], p['sem1_b'][None], p['sem1_w'][H:],
        p['met1_w'][:H], p['met1_b'][None], p['met1_w'][H:2 * H])


def _tc_edge_out_body(g1, g2, dist, sem2, sem2b, metrow, met2, met2b,
                      sem_o, dp_o):
    d = dist[:]
    a1, b1 = _unpack2(g1[:])
    a2, b2 = _unpack2(g2[:])
    hs = _relu(a1 + a2)
    sem_o[:] = jnp.dot(hs, sem2[:], preferred_element_type=F32) + sem2b[:]
    hm = _relu(b1 + b2 + d * metrow[:])
    corr = jnp.dot(hm, met2[:], preferred_element_type=F32) + met2b[:]
    dp_o[:] = d + corr


def _tc_edge_out(g1, g2, dist, p):
    blk = 2048
    g = E // blk
    return pl.pallas_call(
        _tc_edge_out_body,
        grid=(g,),
        in_specs=[
            pl.BlockSpec((blk, H), lambda i: (i, 0)),
            pl.BlockSpec((blk, H), lambda i: (i, 0)),
            pl.BlockSpec((blk, 1), lambda i: (i, 0)),
            pl.BlockSpec((H, C), lambda i: (0, 0)),
            pl.BlockSpec((1, C), lambda i: (0, 0)),
            pl.BlockSpec((1, H), lambda i: (0, 0)),
            pl.BlockSpec((H, 1), lambda i: (0, 0)),
            pl.BlockSpec((1, 1), lambda i: (0, 0)),
        ],
        out_specs=[
            pl.BlockSpec((blk, C), lambda i: (i, 0)),
            pl.BlockSpec((blk, 1), lambda i: (i, 0)),
        ],
        out_shape=(
            jax.ShapeDtypeStruct((E, C), F32),
            jax.ShapeDtypeStruct((E, 1), F32),
        ),
    )(g1, g2, dist, p['sem2_w'], p['sem2_b'][None],
      p['met1_w'][2 * H][None], p['met2_w'], p['met2_b'][None])


_BCH = 2048
_BNCH = E // _BCH


_AQ = 4
_AG = NW // _AQ
_AR = N // _AQ
_ACW = N // _AG


def _sc_build_body(src_h, dst_h, d_h, z_h, avp_o, avp_t,
                   src_v0, dst_v0, d_v0, src_v1, dst_v1, d_v1,
                   sem0, sem1):
    c = lax.axis_index("c")
    s = lax.axis_index("s")
    wid = s * NC + c
    q = lax.rem(wid, _AQ)
    g0 = lax.div(wid, _AQ)
    r_lo = q * _AR
    col_lo = g0 * _ACW
    zc = pltpu.async_copy(z_h, avp_t, sem1)
    bufs = [(src_v0, dst_v0, d_v0, sem0), (src_v1, dst_v1, d_v1, sem1)]

    def stage(k, b):
        sv, dv_, vv, sm = bufs[b]
        return (pltpu.async_copy(src_h.at[pl.ds(k * _BCH, _BCH)], sv, sm),
                pltpu.async_copy(dst_h.at[pl.ds(k * _BCH, _BCH)], dv_, sm),
                pltpu.async_copy(d_h.at[pl.ds(k * _BCH, _BCH)], vv, sm))

    pend = stage(0, 0)
    zc.wait()
    for k in range(_BNCH):
        b = k % 2
        for h in pend:
            h.wait()
        if k + 1 < _BNCH:
            nxt = stage(k + 1, (k + 1) % 2)
        sv, dvv, vv, _sm = bufs[b]

        def body(g, _):
            s16 = sv[pl.ds(g * 16, 16)]
            d16 = dvv[pl.ds(g * 16, 16)]
            dv = vv[pl.ds(g * 16, 16)]
            m2 = ((s16 >= r_lo) & (s16 < r_lo + _AR)
                  & (d16 >= col_lo) & (d16 < col_lo + _ACW))
            flat = jnp.where(m2, (s16 - r_lo) * _ACW + (d16 - col_lo), 0)
            plsc.store_scatter(avp_t, [flat], dv + 1.0, mask=m2)
            return _
        lax.fori_loop(0, _BCH // 16, body, 0)
        if k + 1 < _BNCH:
            pend = nxt
    pltpu.sync_copy(avp_t, avp_o.at[pl.ds(wid * _AR * _ACW, _AR * _ACW)])


def _sp_build(src, dst, d):
    fn = pl.kernel(
        _sc_build_body,
        out_type=jax.ShapeDtypeStruct((NW * _AR * _ACW,), F32),
        mesh=_sc_mesh(),
        scratch_types=[
            pltpu.VMEM((_AR * _ACW,), F32),
            pltpu.VMEM((_BCH,), I32),
            pltpu.VMEM((_BCH,), I32),
            pltpu.VMEM((_BCH,), F32),
            pltpu.VMEM((_BCH,), I32),
            pltpu.VMEM((_BCH,), I32),
            pltpu.VMEM((_BCH,), F32),
            pltpu.SemaphoreType.DMA,
            pltpu.SemaphoreType.DMA,
        ],
        compiler_params=_SC_PARAMS)
    return fn(src, dst, d, jnp.zeros((_AR * _ACW,), F32))


_GCH = 128
_GN = EPW // _GCH


_G2CH = 64
_G2N = EPW // _G2CH


def _sc_edge_gather_body(src_h, dst_h, d_h, p1_h, p2_h, cmsg_h,
                         res_o, wgt_o, cs_o,
                         srcv, dstv, dv, idxf, p1r, p2r, resv, wgtv,
                         rows0, rows1, sem, sem1, sem2, os0, os1):
    c = lax.axis_index("c")
    s = lax.axis_index("s")
    wid = s * NC + c
    base = wid * EPW
    h1 = pltpu.async_copy(src_h.at[pl.ds(base, EPW)], srcv, sem)
    h2 = pltpu.async_copy(dst_h.at[pl.ds(base, EPW)], dstv, sem)
    h3 = pltpu.async_copy(d_h.at[pl.ds(base, EPW)], dv, sem)
    h1.wait(); h2.wait(); h3.wait()

    rbufs = [rows0, rows1]
    gsems = [sem1, sem2]
    osems = [os0, os1]
    gh = [None, None]
    oh = [None, None]
    gh[0] = pltpu.async_copy(cmsg_h.at[srcv.at[pl.ds(0, _G2CH)]], rows0, sem1)

    def fbody(g, _):
        f = srcv[pl.ds(g * 16, 16)] * N + dstv[pl.ds(g * 16, 16)]
        idxf[pl.ds(g * 16, 16)] = f
        return _
    lax.fori_loop(0, EPW // 16, fbody, 0)
    hps = []
    for j in range(_GN):
        hps.append(pltpu.async_copy(
            p1_h.at[idxf.at[pl.ds(j * _GCH, _GCH)]],
            p1r.at[pl.ds(j * _GCH, _GCH)], sem))
        hps.append(pltpu.async_copy(
            p2_h.at[idxf.at[pl.ds(j * _GCH, _GCH)]],
            p2r.at[pl.ds(j * _GCH, _GCH)], sem))
    for h in hps:
        h.wait()

    def rbody(g, _):
        p1g = p1r[pl.ds(g * 16, 16)]
        p2g = p2r[pl.ds(g * 16, 16)]
        dg = dv[pl.ds(g * 16, 16)]
        mean = jnp.where(p2g > 0.0, p1g / jnp.maximum(p2g, 1.0), dg)
        r = jnp.abs(dg - mean)
        resv[pl.ds(g * 16, 16)] = r
        wgtv[pl.ds(g * 16, 16)] = jnp.exp(-r)
        return _
    lax.fori_loop(0, EPW // 16, rbody, 0)
    hr = pltpu.async_copy(resv, res_o.at[pl.ds(base, EPW)], sem)
    hw = pltpu.async_copy(wgtv, wgt_o.at[pl.ds(base, EPW)], sem)
    for j in range(_G2N):
        b = j % 2
        nb = (j + 1) % 2
        if j + 1 < _G2N:
            if oh[nb] is not None:
                oh[nb].wait()
            gh[nb] = pltpu.async_copy(
                cmsg_h.at[srcv.at[pl.ds((j + 1) * _G2CH, _G2CH)]],
                rbufs[nb], gsems[nb])
        gh[b].wait()
        oh[b] = pltpu.async_copy(
            rbufs[b], cs_o.at[pl.ds(base + j * _G2CH, _G2CH)], osems[b])
    oh[0].wait()
    oh[1].wait()
    hr.wait()
    hw.wait()


def _sp_edge_gather(p1m, p2m, cmsg, src, dst, d):
    fn = pl.kernel(
        _sc_edge_gather_body,
        out_type=(jax.ShapeDtypeStruct((E,), F32),
                  jax.ShapeDtypeStruct((E,), F32),
                  jax.ShapeDtypeStruct((E, H), F32)),
        mesh=_sc_mesh(),
        scratch_types=[
            pltpu.VMEM((EPW,), I32),
            pltpu.VMEM((EPW,), I32),
            pltpu.VMEM((EPW,), F32),
            pltpu.VMEM((EPW,), I32),
            pltpu.VMEM((EPW,), F32),
            pltpu.VMEM((EPW,), F32),
            pltpu.VMEM((EPW,), F32),
            pltpu.VMEM((EPW,), F32),
            pltpu.VMEM((_G2CH, H), F32),
            pltpu.VMEM((_G2CH, H), F32),
            pltpu.SemaphoreType.DMA,
            pltpu.SemaphoreType.DMA,
            pltpu.SemaphoreType.DMA,
            pltpu.SemaphoreType.DMA,
            pltpu.SemaphoreType.DMA,
        ],
        compiler_params=_SC_PARAMS)
    res, wgt, cs = fn(src, dst, d, p1m.reshape(N * N), p2m.reshape(N * N),
                      cmsg)
    return res[:, None], wgt[:, None], cs


_FCH = 32
_FN = EPW // _FCH


def _sc_final_gather_body(src_h, dst_h, pr_h, qs_h, g1_o, g2_o,
                          srcv, dstv, bufa0, bufa1, bufb0, bufb1,
                          ga0, ga1, gb0, gb1, oa0, oa1, ob0, ob1, ssem):
    c = lax.axis_index("c")
    s = lax.axis_index("s")
    wid = s * NC + c
    base = wid * EPW
    h1 = pltpu.async_copy(src_h.at[pl.ds(base, EPW)], srcv, ssem)
    h2 = pltpu.async_copy(dst_h.at[pl.ds(base, EPW)], dstv, ssem)
    h1.wait(); h2.wait()
    bufa = [bufa0, bufa1]
    bufb = [bufb0, bufb1]
    gsa = [ga0, ga1]
    gsb = [gb0, gb1]
    osa = [oa0, oa1]
    osb = [ob0, ob1]
    ga = [None, None]
    gb = [None, None]
    oa = [None, None]
    ob = [None, None]
    for j in range(_FN):
        b = j % 2
        if oa[b] is not None:
            oa[b].wait()
            ob[b].wait()
        ga[b] = pltpu.async_copy(pr_h.at[srcv.at[pl.ds(j * _FCH, _FCH)]],
                                 bufa[b], gsa[b])
        gb[b] = pltpu.async_copy(qs_h.at[dstv.at[pl.ds(j * _FCH, _FCH)]],
                                 bufb[b], gsb[b])
        if j >= 1:
            pb = (j - 1) % 2
            ga[pb].wait()
            oa[pb] = pltpu.async_copy(
                bufa[pb], g1_o.at[pl.ds(base + (j - 1) * _FCH, _FCH)],
                osa[pb])
            gb[pb].wait()
            ob[pb] = pltpu.async_copy(
                bufb[pb], g2_o.at[pl.ds(base + (j - 1) * _FCH, _FCH)],
                osb[pb])
    lb = (_FN - 1) % 2
    ga[lb].wait()
    pltpu.sync_copy(bufa[lb], g1_o.at[pl.ds(base + (_FN - 1) * _FCH, _FCH)])
    gb[lb].wait()
    pltpu.sync_copy(bufb[lb], g2_o.at[pl.ds(base + (_FN - 1) * _FCH, _FCH)])
    oa[(_FN - 2) % 2].wait()
    ob[(_FN - 2) % 2].wait()


def _sp_final_gather(pr, qs, src, dst):
    fn = pl.kernel(
        _sc_final_gather_body,
        out_type=(jax.ShapeDtypeStruct((E, H), F32),
                  jax.ShapeDtypeStruct((E, H), F32)),
        mesh=_sc_mesh(),
        scratch_types=[
            pltpu.VMEM((EPW,), I32),
            pltpu.VMEM((EPW,), I32),
            pltpu.VMEM((_FCH, H), F32),
            pltpu.VMEM((_FCH, H), F32),
            pltpu.VMEM((_FCH, H), F32),
            pltpu.VMEM((_FCH, H), F32),
            pltpu.SemaphoreType.DMA,
            pltpu.SemaphoreType.DMA,
            pltpu.SemaphoreType.DMA,
            pltpu.SemaphoreType.DMA,
            pltpu.SemaphoreType.DMA,
            pltpu.SemaphoreType.DMA,
            pltpu.SemaphoreType.DMA,
            pltpu.SemaphoreType.DMA,
            pltpu.SemaphoreType.DMA,
        ],
        compiler_params=_SC_PARAMS)
    return fn(src, dst, pr, qs)


def _prep_params(params):
    p = dict(params)
    mu1 = jnp.zeros((XPAD, H), F32).at[:261].set(params['mu1_w'])
    sg1 = jnp.zeros((XPAD, H), F32).at[:261].set(params['sg1_w'][:261])
    p['mu1_wp'] = mu1
    p['sg1_wp'] = sg1
    p['sg1_row'] = params['sg1_w'][261]
    p['mef_wp'] = jnp.zeros((8, H), F32).at[:4].set(params['msg1_w'][2 * H:])
    return p


def kernel(node_sem, node_bbox, node_depth, edge_index, edge_dist, edge_conf,
           edge_angle, edge_depth_diff, params):
    src = edge_index[0]
    dst = edge_index[1]
    d = edge_dist[:, 0]
    p = _prep_params(params)

    xp = jnp.zeros((N, XPAD), F32).at[:, :261].set(
        jnp.concatenate([node_sem, node_bbox, node_depth], axis=-1))
    efp = jnp.zeros((E, 8), F32).at[:, :4].set(
        jnp.concatenate([edge_dist, edge_conf, edge_angle, edge_depth_diff],
                        axis=-1))

    avpf = _sp_build(src, dst, d)
    mu, cmsg, cntsc = _tc_node1(dst, edge_conf, xp, p)
    cnt = cntsc[:, 0:1]
    p1m, p2m = _tc_resmm(avpf)
    res, wgt, cs = _sp_edge_gather(p1m, p2m, cmsg, src, dst, d)
    agg_aug = _tc_edge_msg(cs, efp, wgt, res, dst, p)
    mu_new, sigma_new, pr, qs = _tc_node2(agg_aug, cnt, mu, p)
    g1, g2 = _sp_final_gather(pr, qs, src, dst)
    sem_logits, dist_pred = _tc_edge_out(g1, g2, edge_dist, p)
    return sem_logits, dist_pred, mu_new, sigma_new, res
